# Initial kernel scaffold; baseline (speedup 1.0000x reference)
import jax
import jax.numpy as jnp
from jax.experimental import pallas as pl


def kernel(gender_idx, ethnicity_idx, W_gender, W_ethnicity):
    raise NotImplementedError("write your pallas kernel here")


# speedup vs baseline: 2.1485x; 2.1485x over previous
import functools

import jax
import jax.numpy as jnp
from jax import lax
from jax.experimental import pallas as pl
from jax.experimental.pallas import tpu as pltpu
from jax.experimental.pallas import tpu_sc as plsc

NG = 2
NE = 8
D = 64
B = 16384
NBINS = NG * NE

NC = 2
NS = 16
L = 16
NW = NC * NS
BPW = B // NW
VPW = BPW // L
CHUNK = 128
NCHUNK = BPW // CHUNK


def _body(g_hbm, e_hbm, wg_hbm, we_hbm,
          emb_hbm, inter_hbm, counts_hbm, comb_hbm,
          g_v, e_v, inter_v, idx_v, rows_v,
          wg_v, we_v, comb_v, cnt_v, acc_v, cntall_v, tot_v,
          cnt_shared, sem):
  cid = lax.axis_index("c")
  sid = lax.axis_index("s")
  wid = sid * NC + cid
  base = wid * BPW

  lane = lax.iota(jnp.int32, L)
  ones = jnp.full((L,), 1, jnp.int32)

  @pl.when(cid == 0)
  def _():
    pltpu.sync_copy(g_hbm.at[pl.ds(base, 2 * BPW)], g_v)
    pltpu.sync_copy(e_hbm.at[pl.ds(base, 2 * BPW)], e_v)

  @pl.when(cid != 0)
  def _():
    pltpu.sync_copy(g_hbm.at[pl.ds(base, BPW)], g_v.at[pl.ds(0, BPW)])
    pltpu.sync_copy(e_hbm.at[pl.ds(base, BPW)], e_v.at[pl.ds(0, BPW)])

  row_off = cid * NBINS
  for j in range(VPW):
    g = g_v[pl.ds(j * L, L)]
    e = e_v[pl.ds(j * L, L)]
    it = g * NE + e
    inter_v[pl.ds(j * L, L)] = it
    idx_v[j // (CHUNK // L), pl.ds((j % (CHUNK // L)) * L, L)] = it + row_off
  pltpu.sync_copy(inter_v, inter_hbm.at[pl.ds(base, BPW)])

  @pl.when(cid == 0)
  def _():
    zeros = jnp.zeros((L,), jnp.int32)
    for r in range(L):
      cnt_v[pl.ds(r * L, L)] = zeros
    lane_base = lane * NBINS
    for j in range(2 * VPW):
      g = g_v[pl.ds(j * L, L)]
      e = e_v[pl.ds(j * L, L)]
      it = g * NE + e
      plsc.addupdate_scatter(cnt_v, [lane_base + it], ones)
    acc = jnp.zeros((L,), jnp.int32)
    for r in range(L):
      acc = acc + cnt_v[pl.ds(r * L, L)]
    acc_v[pl.ds(0, L)] = acc
    pltpu.sync_copy(acc_v, cnt_shared.at[pl.ds(sid * L, L)])

  @pl.when(sid == 0)
  def _():
    pltpu.sync_copy(wg_hbm, wg_v)
    pltpu.sync_copy(we_hbm, we_v)
    for i in range(NBINS):
      for k in range(D // L):
        comb_v[i, pl.ds(k * L, L)] = wg_v[i // NE, pl.ds(k * L, L)]
        comb_v[i, pl.ds(D + k * L, L)] = we_v[i % NE, pl.ds(k * L, L)]
    pltpu.sync_copy(comb_v, comb_hbm.at[pl.ds(row_off, NBINS)])

  plsc.subcore_barrier()

  copies = [
      pltpu.async_copy(comb_hbm.at[idx_v.at[t]],
                       rows_v.at[pl.ds(t * CHUNK, CHUNK)], sem)
      for t in range(NCHUNK)
  ]
  for c in copies:
    c.wait()
  pltpu.sync_copy(rows_v, emb_hbm.at[pl.ds(base, BPW)])

  @pl.when((cid == 0) & (sid == 0))
  def _():
    pltpu.sync_copy(cnt_shared, cntall_v)
    tot = jnp.zeros((L,), jnp.int32)
    for s in range(NS):
      tot = tot + cntall_v[pl.ds(s * L, L)]
    tot_v[pl.ds(0, L)] = tot
    pltpu.sync_copy(tot_v, counts_hbm)


@jax.jit
def _run(gender_idx, ethnicity_idx, W_gender, W_ethnicity):
  mesh = plsc.VectorSubcoreMesh(core_axis_name="c", subcore_axis_name="s")
  emb, inter, counts, _comb = pl.kernel(
      _body,
      out_type=[
          jax.ShapeDtypeStruct((B, NG * D), jnp.float32),
          jax.ShapeDtypeStruct((B,), jnp.int32),
          jax.ShapeDtypeStruct((NBINS,), jnp.int32),
          jax.ShapeDtypeStruct((NC * NBINS, NG * D), jnp.float32),
      ],
      mesh=mesh,
      compiler_params=pltpu.CompilerParams(needs_layout_passes=False),
      scratch_types=[
          pltpu.VMEM((2 * BPW,), jnp.int32),
          pltpu.VMEM((2 * BPW,), jnp.int32),
          pltpu.VMEM((BPW,), jnp.int32),
          pltpu.VMEM((NCHUNK, CHUNK), jnp.int32),
          pltpu.VMEM((BPW, NG * D), jnp.float32),
          pltpu.VMEM((NG, D), jnp.float32),
          pltpu.VMEM((NE, D), jnp.float32),
          pltpu.VMEM((NBINS, NG * D), jnp.float32),
          pltpu.VMEM((L * NBINS,), jnp.int32),
          pltpu.VMEM((L,), jnp.int32),
          pltpu.VMEM((NS * L,), jnp.int32),
          pltpu.VMEM((L,), jnp.int32),
          pltpu.VMEM_SHARED((NS * L,), jnp.int32),
          pltpu.SemaphoreType.DMA,
      ],
  )(gender_idx, ethnicity_idx, W_gender, W_ethnicity)
  return emb, inter, counts


def kernel(gender_idx, ethnicity_idx, W_gender, W_ethnicity):
  return _run(gender_idx.astype(jnp.int32), ethnicity_idx.astype(jnp.int32),
              W_gender, W_ethnicity)


# speedup vs baseline: 3.7724x; 1.7559x over previous
import functools

import jax
import jax.numpy as jnp
from jax import lax
from jax.experimental import pallas as pl
from jax.experimental.pallas import tpu as pltpu
from jax.experimental.pallas import tpu_sc as plsc

NG = 2
NE = 8
D = 64
B = 16384
NBINS = NG * NE

NC = 2
NS = 16
L = 16
NW = NC * NS
BPW = B // NW
VPW = BPW // L
CHUNK = 128
NCHUNK = BPW // CHUNK


def _body(g_hbm, e_hbm, wg_hbm, we_hbm,
          emb_hbm, inter_hbm, counts_hbm, comb_hbm,
          g_v, e_v, inter_v, idx_v, rows_v,
          wg_v, we_v, comb_v, cnt_v, acc_v, cntall_v, tot_v,
          cnt_shared, sem, sem2):
  cid = lax.axis_index("c")
  sid = lax.axis_index("s")
  wid = sid * NC + cid
  base = wid * BPW

  lane = lax.iota(jnp.int32, L)
  ones = jnp.full((L,), 1, jnp.int32)

  @pl.when(cid == 0)
  def _():
    pltpu.sync_copy(g_hbm.at[pl.ds(base, 2 * BPW)], g_v)
    pltpu.sync_copy(e_hbm.at[pl.ds(base, 2 * BPW)], e_v)

  @pl.when(cid != 0)
  def _():
    pltpu.sync_copy(g_hbm.at[pl.ds(base, BPW)], g_v.at[pl.ds(0, BPW)])
    pltpu.sync_copy(e_hbm.at[pl.ds(base, BPW)], e_v.at[pl.ds(0, BPW)])

  row_off = wid * NBINS
  for j in range(VPW):
    g = g_v[pl.ds(j * L, L)]
    e = e_v[pl.ds(j * L, L)]
    it = g * NE + e
    inter_v[pl.ds(j * L, L)] = it
    idx_v[j // (CHUNK // L), pl.ds((j % (CHUNK // L)) * L, L)] = it + row_off
  inter_copy = pltpu.async_copy(inter_v, inter_hbm.at[pl.ds(base, BPW)], sem2)

  pltpu.sync_copy(wg_hbm, wg_v)
  pltpu.sync_copy(we_hbm, we_v)
  for i in range(NBINS):
    for k in range(D // L):
      comb_v[i, pl.ds(k * L, L)] = wg_v[i // NE, pl.ds(k * L, L)]
      comb_v[i, pl.ds(D + k * L, L)] = we_v[i % NE, pl.ds(k * L, L)]
  pltpu.sync_copy(comb_v, comb_hbm.at[pl.ds(row_off, NBINS)])

  gathers = [
      pltpu.async_copy(comb_hbm.at[idx_v.at[t]],
                       rows_v.at[pl.ds(t * CHUNK, CHUNK)], sem)
      for t in range(NCHUNK)
  ]
  writes = []
  for t in range(NCHUNK):
    gathers[t].wait()
    writes.append(
        pltpu.async_copy(rows_v.at[pl.ds(t * CHUNK, CHUNK)],
                         emb_hbm.at[pl.ds(base + t * CHUNK, CHUNK)], sem2))

  @pl.when(cid == 0)
  def _():
    zeros = jnp.zeros((L,), jnp.int32)
    for r in range(L):
      cnt_v[pl.ds(r * L, L)] = zeros
    lane_base = lane * NBINS
    for j in range(2 * VPW):
      g = g_v[pl.ds(j * L, L)]
      e = e_v[pl.ds(j * L, L)]
      it = g * NE + e
      plsc.addupdate_scatter(cnt_v, [lane_base + it], ones)
    acc = jnp.zeros((L,), jnp.int32)
    for r in range(L):
      acc = acc + cnt_v[pl.ds(r * L, L)]
    acc_v[pl.ds(0, L)] = acc
    pltpu.sync_copy(acc_v, cnt_shared.at[pl.ds(sid * L, L)])

  plsc.subcore_barrier()

  @pl.when((cid == 0) & (sid == 0))
  def _():
    pltpu.sync_copy(cnt_shared, cntall_v)
    tot = jnp.zeros((L,), jnp.int32)
    for s in range(NS):
      tot = tot + cntall_v[pl.ds(s * L, L)]
    tot_v[pl.ds(0, L)] = tot
    pltpu.sync_copy(tot_v, counts_hbm)

  for w in writes:
    w.wait()
  inter_copy.wait()


@jax.jit
def _run(gender_idx, ethnicity_idx, W_gender, W_ethnicity):
  mesh = plsc.VectorSubcoreMesh(core_axis_name="c", subcore_axis_name="s")
  emb, inter, counts, _comb = pl.kernel(
      _body,
      out_type=[
          jax.ShapeDtypeStruct((B, NG * D), jnp.float32),
          jax.ShapeDtypeStruct((B,), jnp.int32),
          jax.ShapeDtypeStruct((NBINS,), jnp.int32),
          jax.ShapeDtypeStruct((NW * NBINS, NG * D), jnp.float32),
      ],
      mesh=mesh,
      compiler_params=pltpu.CompilerParams(needs_layout_passes=False),
      scratch_types=[
          pltpu.VMEM((2 * BPW,), jnp.int32),
          pltpu.VMEM((2 * BPW,), jnp.int32),
          pltpu.VMEM((BPW,), jnp.int32),
          pltpu.VMEM((NCHUNK, CHUNK), jnp.int32),
          pltpu.VMEM((BPW, NG * D), jnp.float32),
          pltpu.VMEM((NG, D), jnp.float32),
          pltpu.VMEM((NE, D), jnp.float32),
          pltpu.VMEM((NBINS, NG * D), jnp.float32),
          pltpu.VMEM((L * NBINS,), jnp.int32),
          pltpu.VMEM((L,), jnp.int32),
          pltpu.VMEM((NS * L,), jnp.int32),
          pltpu.VMEM((L,), jnp.int32),
          pltpu.VMEM_SHARED((NS * L,), jnp.int32),
          pltpu.SemaphoreType.DMA,
          pltpu.SemaphoreType.DMA,
      ],
  )(gender_idx, ethnicity_idx, W_gender, W_ethnicity)
  return emb, inter, counts


def kernel(gender_idx, ethnicity_idx, W_gender, W_ethnicity):
  return _run(gender_idx.astype(jnp.int32), ethnicity_idx.astype(jnp.int32),
              W_gender, W_ethnicity)


# speedup vs baseline: 5.5052x; 1.4593x over previous
import functools

import jax
import jax.numpy as jnp
from jax import lax
from jax.experimental import pallas as pl
from jax.experimental.pallas import tpu as pltpu
from jax.experimental.pallas import tpu_sc as plsc

NG = 2
NE = 8
D = 64
B = 16384
NBINS = NG * NE

NC = 2
NS = 16
L = 16
NW = NC * NS
BPW = B // NW
VPW = BPW // L
CHUNK = 128
NCHUNK = BPW // CHUNK


def _body(g_hbm, e_hbm, wg_hbm, we_hbm,
          emb_hbm, inter_hbm, counts_hbm,
          g_v, e_v, inter_v, idx_v, rows_v,
          wg_v, we_v, comb_v,
          cnt_v, acc_v, cntall_v, tot_v,
          cnt_shared, comb_sh, sem, sem2, sem3):
  cid = lax.axis_index("c")
  sid = lax.axis_index("s")
  wid = sid * NC + cid
  base = wid * BPW

  lane = lax.iota(jnp.int32, L)
  ones = jnp.full((L,), 1, jnp.int32)

  wg_copy = pltpu.async_copy(wg_hbm, wg_v, sem3)
  we_copy = pltpu.async_copy(we_hbm, we_v, sem3)

  @pl.when(cid == 0)
  def _():
    pltpu.async_copy(g_hbm.at[pl.ds(base, 2 * BPW)], g_v, sem).wait()
    pltpu.async_copy(e_hbm.at[pl.ds(base, 2 * BPW)], e_v, sem).wait()

  @pl.when(cid != 0)
  def _():
    pltpu.async_copy(g_hbm.at[pl.ds(base, BPW)], g_v.at[pl.ds(0, BPW)],
                     sem).wait()
    pltpu.async_copy(e_hbm.at[pl.ds(base, BPW)], e_v.at[pl.ds(0, BPW)],
                     sem).wait()

  def inter_body(j, _):
    g = g_v[pl.ds(j * L, L)]
    e = e_v[pl.ds(j * L, L)]
    it = g * NE + e
    inter_v[pl.ds(j * L, L)] = it
    idx_v[pl.ds(j * L, L)] = it
    return 0

  lax.fori_loop(0, VPW, inter_body, 0)
  inter_copy = pltpu.async_copy(inter_v, inter_hbm.at[pl.ds(base, BPW)], sem2)

  @pl.when(sid == 0)
  def _():
    wg_copy.wait()
    we_copy.wait()
    for i in range(NBINS):
      for k in range(D // L):
        comb_v[i, pl.ds(k * L, L)] = wg_v[i // NE, pl.ds(k * L, L)]
        comb_v[i, pl.ds(D + k * L, L)] = we_v[i % NE, pl.ds(k * L, L)]
    pltpu.sync_copy(comb_v, comb_sh)

  @pl.when(sid != 0)
  def _():
    wg_copy.wait()
    we_copy.wait()

  plsc.subcore_barrier()

  gathers = [
      pltpu.async_copy(comb_sh.at[idx_v.at[pl.ds(t * CHUNK, CHUNK)]],
                       rows_v.at[pl.ds(t * CHUNK, CHUNK)], sem)
      for t in range(NCHUNK)
  ]
  writes = []
  for t in range(NCHUNK):
    gathers[t].wait()
    writes.append(
        pltpu.async_copy(rows_v.at[pl.ds(t * CHUNK, CHUNK)],
                         emb_hbm.at[pl.ds(base + t * CHUNK, CHUNK)], sem2))

  @pl.when(cid == 0)
  def _():
    zeros = jnp.zeros((L,), jnp.int32)
    for r in range(L):
      cnt_v[pl.ds(r * L, L)] = zeros
    lane_base = lane * NBINS

    def cnt_body(j, _):
      g = g_v[pl.ds(j * L, L)]
      e = e_v[pl.ds(j * L, L)]
      plsc.addupdate_scatter(cnt_v, [lane_base + g * NE + e], ones)
      return 0

    lax.fori_loop(0, 2 * VPW, cnt_body, 0)
    acc = jnp.zeros((L,), jnp.int32)
    for r in range(L):
      acc = acc + cnt_v[pl.ds(r * L, L)]
    acc_v[pl.ds(0, L)] = acc
    pltpu.sync_copy(acc_v, cnt_shared.at[pl.ds(sid * L, L)])

  plsc.subcore_barrier()

  @pl.when((cid == 0) & (sid == 0))
  def _():
    pltpu.sync_copy(cnt_shared, cntall_v)
    tot = jnp.zeros((L,), jnp.int32)
    for s in range(NS):
      tot = tot + cntall_v[pl.ds(s * L, L)]
    tot_v[pl.ds(0, L)] = tot
    pltpu.sync_copy(tot_v, counts_hbm)

  for w in writes:
    w.wait()
  inter_copy.wait()


@jax.jit
def _run(gender_idx, ethnicity_idx, W_gender, W_ethnicity):
  mesh = plsc.VectorSubcoreMesh(core_axis_name="c", subcore_axis_name="s")
  emb, inter, counts = pl.kernel(
      _body,
      out_type=[
          jax.ShapeDtypeStruct((B, NG * D), jnp.float32),
          jax.ShapeDtypeStruct((B,), jnp.int32),
          jax.ShapeDtypeStruct((NBINS,), jnp.int32),
      ],
      mesh=mesh,
      compiler_params=pltpu.CompilerParams(needs_layout_passes=False),
      scratch_types=[
          pltpu.VMEM((2 * BPW,), jnp.int32),
          pltpu.VMEM((2 * BPW,), jnp.int32),
          pltpu.VMEM((BPW,), jnp.int32),
          pltpu.VMEM((BPW,), jnp.int32),
          pltpu.VMEM((BPW, NG * D), jnp.float32),
          pltpu.VMEM((NG, D), jnp.float32),
          pltpu.VMEM((NE, D), jnp.float32),
          pltpu.VMEM((NBINS, NG * D), jnp.float32),
          pltpu.VMEM((L * NBINS,), jnp.int32),
          pltpu.VMEM((L,), jnp.int32),
          pltpu.VMEM((NS * L,), jnp.int32),
          pltpu.VMEM((L,), jnp.int32),
          pltpu.VMEM_SHARED((NS * L,), jnp.int32),
          pltpu.VMEM_SHARED((NBINS, NG * D), jnp.float32),
          pltpu.SemaphoreType.DMA,
          pltpu.SemaphoreType.DMA,
          pltpu.SemaphoreType.DMA,
      ],
  )(gender_idx, ethnicity_idx, W_gender, W_ethnicity)
  return emb, inter, counts


def kernel(gender_idx, ethnicity_idx, W_gender, W_ethnicity):
  return _run(gender_idx.astype(jnp.int32), ethnicity_idx.astype(jnp.int32),
              W_gender, W_ethnicity)


# speedup vs baseline: 5.5203x; 1.0027x over previous
import functools

import jax
import jax.numpy as jnp
from jax import lax
from jax.experimental import pallas as pl
from jax.experimental.pallas import tpu as pltpu
from jax.experimental.pallas import tpu_sc as plsc

NG = 2
NE = 8
D = 64
B = 16384
NBINS = NG * NE

NC = 2
NS = 16
L = 16
NW = NC * NS
BPW = B // NW
VPW = BPW // L
CHUNK = 128
NCHUNK = BPW // CHUNK


def _body(g_hbm, e_hbm, wg_hbm, we_hbm,
          emb_hbm, inter_hbm, counts_hbm,
          g_v, e_v, inter_v, idx_v, rows_v,
          wg_v, we_v, comb_v,
          cnt_v, acc_v, cntall_v, tot_v,
          cnt_shared, comb_sh, sem, sem2, sem3):
  cid = lax.axis_index("c")
  sid = lax.axis_index("s")
  wid = sid * NC + cid
  base = wid * BPW

  lane = lax.iota(jnp.int32, L)
  ones = jnp.full((L,), 1, jnp.int32)

  wg_copy = pltpu.async_copy(wg_hbm, wg_v, sem3)
  we_copy = pltpu.async_copy(we_hbm, we_v, sem3)

  @pl.when(cid == 0)
  def _():
    pltpu.async_copy(g_hbm.at[pl.ds(base, 2 * BPW)], g_v, sem).wait()
    pltpu.async_copy(e_hbm.at[pl.ds(base, 2 * BPW)], e_v, sem).wait()

  @pl.when(cid != 0)
  def _():
    pltpu.async_copy(g_hbm.at[pl.ds(base, BPW)], g_v.at[pl.ds(0, BPW)],
                     sem).wait()
    pltpu.async_copy(e_hbm.at[pl.ds(base, BPW)], e_v.at[pl.ds(0, BPW)],
                     sem).wait()

  def inter_body(j, _):
    g = g_v[pl.ds(j * L, L)]
    e = e_v[pl.ds(j * L, L)]
    it = g * NE + e
    inter_v[pl.ds(j * L, L)] = it
    idx_v[pl.ds(j * L, L)] = it
    return 0

  lax.fori_loop(0, VPW, inter_body, 0)
  inter_copy = pltpu.async_copy(inter_v, inter_hbm.at[pl.ds(base, BPW)], sem2)

  @pl.when(sid == 0)
  def _():
    wg_copy.wait()
    we_copy.wait()

    def build_body(i, _):
      for k in range(D // L):
        comb_v[i, pl.ds(k * L, L)] = wg_v[i // NE, pl.ds(k * L, L)]
        comb_v[i, pl.ds(D + k * L, L)] = we_v[i % NE, pl.ds(k * L, L)]
      return 0

    lax.fori_loop(0, NBINS, build_body, 0)
    pltpu.sync_copy(comb_v, comb_sh)

  @pl.when(sid != 0)
  def _():
    wg_copy.wait()
    we_copy.wait()

  plsc.subcore_barrier()

  gathers = [
      pltpu.async_copy(comb_sh.at[idx_v.at[pl.ds(t * CHUNK, CHUNK)]],
                       rows_v.at[pl.ds(t * CHUNK, CHUNK)], sem)
      for t in range(NCHUNK)
  ]
  writes = []
  for t in range(NCHUNK):
    gathers[t].wait()
    writes.append(
        pltpu.async_copy(rows_v.at[pl.ds(t * CHUNK, CHUNK)],
                         emb_hbm.at[pl.ds(base + t * CHUNK, CHUNK)], sem2))

  @pl.when(cid == 0)
  def _():
    zeros = jnp.zeros((L,), jnp.int32)

    def zero_body(r, _):
      cnt_v[pl.ds(r * L, L)] = zeros
      return 0

    lax.fori_loop(0, L, zero_body, 0)
    lane_base = lane * NBINS

    def cnt_body(j, _):
      g = g_v[pl.ds(j * L, L)]
      e = e_v[pl.ds(j * L, L)]
      plsc.addupdate_scatter(cnt_v, [lane_base + g * NE + e], ones)
      return 0

    lax.fori_loop(0, 2 * VPW, cnt_body, 0)
    acc = lax.fori_loop(
        0, L, lambda r, a: a + cnt_v[pl.ds(r * L, L)],
        jnp.zeros((L,), jnp.int32))
    acc_v[pl.ds(0, L)] = acc
    pltpu.sync_copy(acc_v, cnt_shared.at[pl.ds(sid * L, L)])

  plsc.subcore_barrier()

  @pl.when((cid == 0) & (sid == 0))
  def _():
    pltpu.sync_copy(cnt_shared, cntall_v)
    tot = lax.fori_loop(
        0, NS, lambda r, a: a + cntall_v[pl.ds(r * L, L)],
        jnp.zeros((L,), jnp.int32))
    tot_v[pl.ds(0, L)] = tot
    pltpu.sync_copy(tot_v, counts_hbm)

  for w in writes:
    w.wait()
  inter_copy.wait()


@jax.jit
def _run(gender_idx, ethnicity_idx, W_gender, W_ethnicity):
  mesh = plsc.VectorSubcoreMesh(core_axis_name="c", subcore_axis_name="s")
  emb, inter, counts = pl.kernel(
      _body,
      out_type=[
          jax.ShapeDtypeStruct((B, NG * D), jnp.float32),
          jax.ShapeDtypeStruct((B,), jnp.int32),
          jax.ShapeDtypeStruct((NBINS,), jnp.int32),
      ],
      mesh=mesh,
      compiler_params=pltpu.CompilerParams(needs_layout_passes=False),
      scratch_types=[
          pltpu.VMEM((2 * BPW,), jnp.int32),
          pltpu.VMEM((2 * BPW,), jnp.int32),
          pltpu.VMEM((BPW,), jnp.int32),
          pltpu.VMEM((BPW,), jnp.int32),
          pltpu.VMEM((BPW, NG * D), jnp.float32),
          pltpu.VMEM((NG, D), jnp.float32),
          pltpu.VMEM((NE, D), jnp.float32),
          pltpu.VMEM((NBINS, NG * D), jnp.float32),
          pltpu.VMEM((L * NBINS,), jnp.int32),
          pltpu.VMEM((L,), jnp.int32),
          pltpu.VMEM((NS * L,), jnp.int32),
          pltpu.VMEM((L,), jnp.int32),
          pltpu.VMEM_SHARED((NS * L,), jnp.int32),
          pltpu.VMEM_SHARED((NBINS, NG * D), jnp.float32),
          pltpu.SemaphoreType.DMA,
          pltpu.SemaphoreType.DMA,
          pltpu.SemaphoreType.DMA,
      ],
  )(gender_idx, ethnicity_idx, W_gender, W_ethnicity)
  return emb, inter, counts


def kernel(gender_idx, ethnicity_idx, W_gender, W_ethnicity):
  return _run(gender_idx.astype(jnp.int32), ethnicity_idx.astype(jnp.int32),
              W_gender, W_ethnicity)
